# Initial kernel scaffold; baseline (speedup 1.0000x reference)
import jax
import jax.numpy as jnp
from jax.experimental import pallas as pl


def kernel(x, signal_features, edge_index, edge_type, node_emb, W1_rel, W1_root, b1, W2_rel, W2_root, b2):
    raise NotImplementedError("write your pallas kernel here")


# speedup vs baseline: 10.4987x; 10.4987x over previous
import functools

import jax
import jax.numpy as jnp
from jax.experimental import pallas as pl

N = 17
NP = 24
E = 74
R = 8
NR = N * R


def _onehot_f32(shape, dim, idx_row):
    io = jax.lax.broadcasted_iota(jnp.int32, shape, dim)
    return (io == idx_row).astype(jnp.float32)


def _rgcn_body(x_ref, sf_ref, src_ref, dst_ref, et_ref, emb_ref,
               w1r_ref, w1root_ref, b1_ref, w2r_ref, w2root_ref, b2_ref,
               out_ref):
    f32 = jnp.float32

    emb = emb_ref[...]
    x = x_ref[...]
    g = _onehot_f32((N, N), 0, x)
    h17 = jax.lax.dot_general(g, emb, (((0,), (0,)), ((), ())),
                              preferred_element_type=f32)
    h = jnp.concatenate([sf_ref[...], h17[6:, :]], axis=0)
    hp = jnp.concatenate([h, jnp.zeros((NP - N, h.shape[1]), f32)], axis=0)

    src = src_ref[...]
    dst = dst_ref[...]
    et = et_ref[...]
    k = dst * R + et
    kc = _onehot_f32((NR, E), 0, k)
    counts = jnp.sum(kc, axis=1, keepdims=True)
    inv = 1.0 / jnp.maximum(counts, 1.0)
    norm = jnp.sum(kc * inv, axis=0, keepdims=True)
    rd = et * NP + dst
    u = _onehot_f32((R * NP, E), 0, rd) * norm
    s_t = _onehot_f32((NP, E), 0, src)
    a = jax.lax.dot_general(u, s_t, (((1,), (1,)), ((), ())),
                            preferred_element_type=f32)

    def layer(hin, wr_ref, wroot_ref, b_ref):
        m = jnp.dot(a, hin, preferred_element_type=f32)
        acc = jnp.dot(hin, wroot_ref[...], preferred_element_type=f32)
        for r in range(R):
            acc += jnp.dot(m[r * NP:(r + 1) * NP, :], wr_ref[r],
                           preferred_element_type=f32)
        return acc + b_ref[...]

    h1 = jax.nn.relu(layer(hp, w1r_ref, w1root_ref, b1_ref))
    h2 = layer(h1, w2r_ref, w2root_ref, b2_ref)
    out_ref[...] = h2[:N, :]


@jax.jit
def kernel(x, signal_features, edge_index, edge_type, node_emb,
           W1_rel, W1_root, b1, W2_rel, W2_root, b2):
    call = pl.pallas_call(
        _rgcn_body,
        out_shape=jax.ShapeDtypeStruct((N, node_emb.shape[1]), jnp.float32),
    )
    return call(
        x.astype(jnp.int32).reshape(1, N),
        signal_features,
        edge_index[0].reshape(1, E),
        edge_index[1].reshape(1, E),
        edge_type.reshape(1, E),
        node_emb,
        W1_rel, W1_root, b1.reshape(1, -1),
        W2_rel, W2_root, b2.reshape(1, -1),
    )
